# baseline (speedup 1.0000x reference)
import functools

import jax
import jax.numpy as jnp
from jax import lax
from jax.experimental import pallas as pl
from jax.experimental.pallas import tpu as pltpu
from jax.experimental.pallas import tpu_sc as plsc

_K = 1024
_D = 256
_T = 1024
_BETA = 0.25
_BB = 2
_NC = 2
_NS = 16
_NW = _NC * _NS
_CH = 64


def _vq_tc_kernel(has_alias, z_ref, zsq_ref, emb_ref, esq_ref, *rest):
    if has_alias:
        rest = rest[2:]
    (dist_ref, idx_ref, zst_ref, counts_ref, sse_ref,
     acc_counts, acc_sse) = rest
    step = pl.program_id(0)
    nsteps = pl.num_programs(0)
    emb = emb_ref[...]
    esq = esq_ref[...]

    @pl.when(step == 0)
    def _init():
        acc_counts[...] = jnp.zeros_like(acc_counts)
        acc_sse[...] = jnp.zeros_like(acc_sse)

    t = z_ref.shape[2]
    for j in range(_BB):
        z = z_ref[j]
        zsq = zsq_ref[pl.ds(j * t, t), :]
        prod = jax.lax.dot_general(z, emb, (((0,), (1,)), ((), ())),
                                   preferred_element_type=jnp.float32)
        dist = (zsq + esq) - 2.0 * prod
        dist_ref[pl.ds(j * t, t), :] = dist

        rowmin = jnp.min(dist, axis=1, keepdims=True)
        iota_k = jax.lax.broadcasted_iota(jnp.int32, dist.shape, 1)
        tied = dist == rowmin
        idx = jnp.min(jnp.where(tied, iota_k, _K), axis=1).astype(jnp.int32)
        idx_ref[j, 0, :] = idx
        onehot = (iota_k == idx[:, None]).astype(jnp.float32)

        qT = jax.lax.dot_general(emb, onehot, (((0,), (1,)), ((), ())),
                                 preferred_element_type=jnp.float32)
        zst_ref[j] = qT

        acc_counts[...] += jnp.sum(onehot, axis=0, keepdims=True)
        acc_sse[...] += jnp.sum(rowmin).reshape(1, 1)

    @pl.when(step == nsteps - 1)
    def _finalize():
        counts_ref[...] = acc_counts[...]
        sse_ref[...] = acc_sse[...]


def _tc_half(z3, zsq, embedding, esq, half, nsteps, dist_in=None, zst_in=None):
    b2, d, t = z3.shape
    n = 2 * nsteps * _BB * t
    out_shapes = (
        jax.ShapeDtypeStruct((n, _K), jnp.float32),
        jax.ShapeDtypeStruct((b2, 1, t), jnp.int32),
        jax.ShapeDtypeStruct((2 * b2, d, t), jnp.float32),
        jax.ShapeDtypeStruct((1, _K), jnp.float32),
        jax.ShapeDtypeStruct((1, 1), jnp.float32),
    )
    base = half * nsteps
    out_specs = (
        pl.BlockSpec((_BB * t, _K), lambda i: (base + i, 0)),
        pl.BlockSpec((_BB, 1, t), lambda i: (i, 0, 0)),
        pl.BlockSpec((_BB, d, t), lambda i: (base + i, 0, 0)),
        pl.BlockSpec((1, _K), lambda i: (0, 0)),
        pl.BlockSpec((1, 1), lambda i: (0, 0)),
    )
    in_specs = [
        pl.BlockSpec((_BB, d, t), lambda i: (i, 0, 0)),
        pl.BlockSpec((_BB * t, 1), lambda i: (i, 0)),
        pl.BlockSpec((_K, d), lambda i: (0, 0)),
        pl.BlockSpec((1, _K), lambda i: (0, 0)),
    ]
    args = [z3, zsq, embedding, esq]
    aliases = {}
    if dist_in is not None:
        in_specs.append(pl.BlockSpec(memory_space=pl.ANY))
        in_specs.append(pl.BlockSpec(memory_space=pl.ANY))
        args.append(dist_in)
        args.append(zst_in)
        aliases = {4: 0, 5: 2}
    return pl.pallas_call(
        functools.partial(_vq_tc_kernel, dist_in is not None),
        grid=(nsteps,),
        in_specs=in_specs,
        out_specs=out_specs,
        out_shape=out_shapes,
        input_output_aliases=aliases,
        scratch_shapes=[pltpu.VMEM((1, _K), jnp.float32),
                        pltpu.VMEM((1, 1), jnp.float32)],
        compiler_params=pltpu.CompilerParams(
            dimension_semantics=("arbitrary",)),
    )(*args)


def _sc_onehot_body(row_base, tokens, idx_hbm, zeros_hbm, onehot_hbm,
                    idx_v, buf_v):
    b_per_w = tokens // _NW
    nchunks = b_per_w // _CH
    wid = lax.axis_index("s") * _NC + lax.axis_index("c")
    base = wid * b_per_w
    pltpu.sync_copy(idx_hbm.at[pl.ds(base, b_per_w)], idx_v)

    zeros16 = jnp.zeros((16,), jnp.float32)
    ones16 = jnp.ones((16,), jnp.float32)
    pltpu.sync_copy(zeros_hbm, buf_v)

    for c in range(nchunks):
        for g in range(_CH // 16):
            t16 = lax.iota(jnp.int32, 16) + (g * 16)
            i16 = idx_v[pl.ds(c * _CH + g * 16, 16)]
            plsc.store_scatter(buf_v, [t16 * _K + i16], ones16)
        pltpu.sync_copy(
            buf_v,
            onehot_hbm.at[pl.ds((row_base + base + c * _CH) * _K, _CH * _K)])
        if c + 1 < nchunks:
            for g in range(_CH // 16):
                t16 = lax.iota(jnp.int32, 16) + (g * 16)
                i16 = idx_v[pl.ds(c * _CH + g * 16, 16)]
                plsc.store_scatter(buf_v, [t16 * _K + i16], zeros16)


def _sc_mesh():
    return plsc.VectorSubcoreMesh(core_axis_name="c", subcore_axis_name="s",
                                  num_cores=_NC, num_subcores=_NS)


def _sc_onehot_first(idx_flat, n, tokens):
    b_per_w = tokens // _NW
    kern = pl.kernel(
        functools.partial(_sc_onehot_body, 0, tokens),
        out_type=jax.ShapeDtypeStruct((n * _K,), jnp.float32),
        mesh=_sc_mesh(),
        scratch_types=[pltpu.VMEM((b_per_w,), jnp.int32),
                       pltpu.VMEM((_CH * _K,), jnp.float32)],
        compiler_params=pltpu.CompilerParams(needs_layout_passes=False),
    )
    return kern(idx_flat, jnp.zeros((_CH * _K,), jnp.float32))


def _sc_onehot_second(idx_flat, onehot_ref, row_base, tokens):
    b_per_w = tokens // _NW
    kern = pl.kernel(
        functools.partial(_sc_onehot_body, row_base, tokens),
        out_type=(),
        mesh=_sc_mesh(),
        scratch_types=[pltpu.VMEM((b_per_w,), jnp.int32),
                       pltpu.VMEM((_CH * _K,), jnp.float32)],
        compiler_params=pltpu.CompilerParams(needs_layout_passes=False),
    )
    kern(idx_flat, jnp.zeros((_CH * _K,), jnp.float32), onehot_ref)


def _combine_kernel(n_tokens, counts_ref, sse_ref, loss_ref, perp_ref):
    sse = jnp.sum(sse_ref[...])
    loss_ref[...] = ((1.0 + _BETA) * sse / (n_tokens * _D)).reshape(1, 1)
    counts = jnp.sum(counts_ref[...], axis=0, keepdims=True)
    p = counts / n_tokens
    perp_ref[...] = jnp.exp(-jnp.sum(p * jnp.log(p + 1e-10))).reshape(1, 1)


def kernel(z, embedding):
    b, d, h, w = z.shape
    k = embedding.shape[0]
    t = h * w
    n = b * t
    bh = b // 2
    nsteps = bh // _BB
    z3 = z.reshape(b, d, t)
    z_flat = jnp.transpose(z, (0, 2, 3, 1)).reshape(-1, d)
    zsq = jnp.sum(z_flat ** 2, axis=1, keepdims=True)
    esq = jnp.sum(embedding ** 2, axis=1)[None, :]

    dist0, idx0, zst0, counts0, sse0 = _tc_half(
        z3[:bh], zsq[:bh * t], embedding, esq, 0, nsteps)
    onehot0 = _sc_onehot_first(idx0.reshape(bh * t), n, bh * t)
    dist, idx1, zst, counts1, sse1 = _tc_half(
        z3[bh:], zsq[bh * t:], embedding, esq, 1, nsteps,
        dist_in=dist0, zst_in=zst0)
    oh_ref = jax.new_ref(onehot0)
    _sc_onehot_second(idx1.reshape(bh * t), oh_ref, bh * t, bh * t)
    onehot = oh_ref[...].reshape(n, k)

    loss, perp = pl.pallas_call(
        functools.partial(_combine_kernel, n),
        out_shape=(jax.ShapeDtypeStruct((1, 1), jnp.float32),
                   jax.ShapeDtypeStruct((1, 1), jnp.float32)),
    )(jnp.concatenate([counts0, counts1], axis=0),
      jnp.concatenate([sse0, sse1], axis=0))

    z_quantized_st = zst.reshape(b, d, h, w)
    encoding_indices = jnp.concatenate([idx0, idx1], axis=0).reshape(n)
    return (z_quantized_st, loss[0, 0], perp[0, 0], onehot,
            encoding_indices, dist)


# speedup vs baseline: 2.1842x; 2.1842x over previous
import jax
import jax.numpy as jnp
from jax.experimental import pallas as pl
from jax.experimental.pallas import tpu as pltpu

_K = 1024
_D = 256
_BETA = 0.25
_BB = 2


def _vq_kernel(z_ref, zsq_ref, emb_ref, esq_ref,
               dist_ref, onehot_ref, idx_ref, zst_ref, loss_ref, perp_ref,
               counts_ref, sse_ref):
    step = pl.program_id(0)
    nsteps = pl.num_programs(0)
    emb = emb_ref[...]
    esq = esq_ref[...]

    @pl.when(step == 0)
    def _init():
        counts_ref[...] = jnp.zeros_like(counts_ref)
        sse_ref[...] = jnp.zeros_like(sse_ref)

    t = z_ref.shape[2]
    for j in range(_BB):
        z = z_ref[j]
        zsq = zsq_ref[pl.ds(j * t, t), :]
        prod = jax.lax.dot_general(z, emb, (((0,), (1,)), ((), ())),
                                   preferred_element_type=jnp.float32)
        dist = (zsq + esq) - 2.0 * prod
        dist_ref[pl.ds(j * t, t), :] = dist

        rowmin = jnp.min(dist, axis=1, keepdims=True)
        iota_k = jax.lax.broadcasted_iota(jnp.int32, dist.shape, 1)
        tied = dist == rowmin
        idx = jnp.min(jnp.where(tied, iota_k, _K), axis=1).astype(jnp.int32)
        idx_ref[j, 0, :] = idx
        onehot = (iota_k == idx[:, None]).astype(jnp.float32)
        onehot_ref[pl.ds(j * t, t), :] = onehot

        qT = jax.lax.dot_general(emb, onehot, (((0,), (1,)), ((), ())),
                                 preferred_element_type=jnp.float32)
        zst_ref[j] = qT

        counts_ref[...] += jnp.sum(onehot, axis=0, keepdims=True)
        sse_ref[...] += jnp.sum(rowmin).reshape(1, 1)

    @pl.when(step == nsteps - 1)
    def _finalize():
        n_tokens = nsteps * _BB * t
        sse = sse_ref[...]
        loss_ref[...] = (1.0 + _BETA) * sse / (n_tokens * _D)
        p = counts_ref[...] / n_tokens
        perp_ref[...] = jnp.exp(-jnp.sum(p * jnp.log(p + 1e-10))).reshape(1, 1)


def kernel(z, embedding):
    b, d, h, w = z.shape
    k = embedding.shape[0]
    t = h * w
    n = b * t
    z3 = z.reshape(b, d, t)
    z_flat = jnp.transpose(z, (0, 2, 3, 1)).reshape(-1, d)
    zsq = jnp.sum(z_flat ** 2, axis=1, keepdims=True)
    esq = jnp.sum(embedding ** 2, axis=1)[None, :]

    grid = (b // _BB,)
    out_shapes = (
        jax.ShapeDtypeStruct((n, k), jnp.float32),
        jax.ShapeDtypeStruct((n, k), jnp.float32),
        jax.ShapeDtypeStruct((b, 1, t), jnp.int32),
        jax.ShapeDtypeStruct((b, d, t), jnp.float32),
        jax.ShapeDtypeStruct((1, 1), jnp.float32),
        jax.ShapeDtypeStruct((1, 1), jnp.float32),
    )
    out_specs = (
        pl.BlockSpec((_BB * t, k), lambda i: (i, 0)),
        pl.BlockSpec((_BB * t, k), lambda i: (i, 0)),
        pl.BlockSpec((_BB, 1, t), lambda i: (i, 0, 0)),
        pl.BlockSpec((_BB, d, t), lambda i: (i, 0, 0)),
        pl.BlockSpec((1, 1), lambda i: (0, 0)),
        pl.BlockSpec((1, 1), lambda i: (0, 0)),
    )
    in_specs = (
        pl.BlockSpec((_BB, d, t), lambda i: (i, 0, 0)),
        pl.BlockSpec((_BB * t, 1), lambda i: (i, 0)),
        pl.BlockSpec((k, d), lambda i: (0, 0)),
        pl.BlockSpec((1, k), lambda i: (0, 0)),
    )
    dist, onehot, idx, zst, loss, perp = pl.pallas_call(
        _vq_kernel,
        grid=grid,
        in_specs=in_specs,
        out_specs=out_specs,
        out_shape=out_shapes,
        scratch_shapes=[pltpu.VMEM((1, k), jnp.float32),
                        pltpu.VMEM((1, 1), jnp.float32)],
        compiler_params=pltpu.CompilerParams(
            dimension_semantics=("arbitrary",)),
    )(z3, zsq, embedding, esq)

    z_quantized_st = zst.reshape(b, d, h, w)
    encoding_indices = idx.reshape(n)
    return (z_quantized_st, loss[0, 0], perp[0, 0], onehot,
            encoding_indices, dist)
